# baseline (speedup 1.0000x reference)
import jax
import jax.numpy as jnp
from jax.experimental import pallas as pl
from jax.experimental.pallas import tpu as pltpu

IN_FEATURES = 1024
N_CLASSES = 100000
C1 = 2000
C2 = 10000
N_TOKENS = 1024
BLK = 512
NBLK = (N_CLASSES + BLK - 1) // BLK


def _flash_kernel(x_ref, w_ref, b_ref, tgt_ref, tv_ref, tb_ref,
                  out_ref, loss_ref,
                  m0, s0, m1, s1, m2, s2, pk, tl):
    blk = pl.program_id(0)
    nblk = pl.num_programs(0)

    @pl.when(blk == 0)
    def _init():
        tlog = jax.lax.dot_general(
            x_ref[...], tv_ref[...], (((1,), (1,)), ((), ())),
            preferred_element_type=jnp.float32) + tb_ref[...]
        tl[...] = tlog
        tmax = jnp.max(tlog, axis=1, keepdims=True)
        tl_sum = jnp.sum(jnp.exp(tlog - tmax), axis=1, keepdims=True)
        m0[...] = tmax
        s0[...] = tl_sum
        m1[...] = jnp.full((N_TOKENS, 1), -1e30, jnp.float32)
        s1[...] = jnp.zeros((N_TOKENS, 1), jnp.float32)
        m2[...] = jnp.full((N_TOKENS, 1), -1e30, jnp.float32)
        s2[...] = jnp.zeros((N_TOKENS, 1), jnp.float32)
        pk[...] = jnp.zeros((N_TOKENS, 1), jnp.float32)

    logits = jax.lax.dot_general(
        x_ref[...].astype(jnp.bfloat16), w_ref[...].astype(jnp.bfloat16),
        (((1,), (1,)), ((), ())),
        preferred_element_type=jnp.float32)
    logits = logits + b_ref[0]

    cls = blk * BLK + jax.lax.broadcasted_iota(jnp.int32, (1, BLK), 1)
    in0 = cls < C1
    in1 = (cls >= C1) & (cls < C2)
    in2 = (cls >= C2) & (cls < N_CLASSES)
    neg = jnp.float32(-1e30)

    bm0 = jnp.max(jnp.where(in0, logits, neg), axis=1, keepdims=True)
    bm1 = jnp.max(jnp.where(in1, logits, neg), axis=1, keepdims=True)
    bm2 = jnp.max(jnp.where(in2, logits, neg), axis=1, keepdims=True)
    mo0, mo1, mo2 = m0[...], m1[...], m2[...]
    mn0 = jnp.maximum(mo0, bm0)
    mn1 = jnp.maximum(mo1, bm1)
    mn2 = jnp.maximum(mo2, bm2)

    off = jnp.where(in0, mn0, jnp.where(in1, mn1, mn2))
    e = jnp.exp(logits - off)
    s0[...] = s0[...] * jnp.exp(mo0 - mn0) + jnp.sum(
        jnp.where(in0, e, 0.0), axis=1, keepdims=True)
    s1[...] = s1[...] * jnp.exp(mo1 - mn1) + jnp.sum(
        jnp.where(in1, e, 0.0), axis=1, keepdims=True)
    s2[...] = s2[...] * jnp.exp(mo2 - mn2) + jnp.sum(
        jnp.where(in2, e, 0.0), axis=1, keepdims=True)
    m0[...] = mn0
    m1[...] = mn1
    m2[...] = mn2

    hit = cls == tgt_ref[...]
    pk[...] = pk[...] + jnp.sum(jnp.where(hit, logits, 0.0),
                                axis=1, keepdims=True)

    @pl.when(blk == nblk - 1)
    def _fini():
        tgt = tgt_ref[...]
        lse0 = m0[...] + jnp.log(s0[...])
        lse1 = m1[...] + jnp.log(s1[...])
        lse2 = m2[...] + jnp.log(s2[...])
        is0 = tgt < C1
        is1 = (tgt >= C1) & (tgt < C2)
        p = pk[...]
        tlv = tl[...]
        head_pick = jnp.where(is0, p, jnp.where(is1, tlv[:, 0:1], tlv[:, 1:2]))
        tail_part = jnp.where(is0, 0.0, p - jnp.where(is1, lse1, lse2))
        out = head_pick - lse0 + tail_part
        out_ref[...] = out
        loss_ref[...] = jnp.full((1, 1), 0.0, jnp.float32) - jnp.mean(out)


def kernel(x, target, weight, bias, tail_vectors, tail_bias):
    bias_p = jnp.pad(bias, (0, NBLK * BLK - N_CLASSES)).reshape(NBLK, 1, BLK)
    tgt2 = target.astype(jnp.int32).reshape(N_TOKENS, 1)
    tb2 = tail_bias.reshape(1, 2)
    out, loss = pl.pallas_call(
        _flash_kernel,
        grid=(NBLK,),
        in_specs=[
            pl.BlockSpec((N_TOKENS, IN_FEATURES), lambda b: (0, 0)),
            pl.BlockSpec((BLK, IN_FEATURES), lambda b: (b, 0)),
            pl.BlockSpec((1, 1, BLK), lambda b: (b, 0, 0)),
            pl.BlockSpec((N_TOKENS, 1), lambda b: (0, 0)),
            pl.BlockSpec((2, IN_FEATURES), lambda b: (0, 0)),
            pl.BlockSpec((1, 2), lambda b: (0, 0)),
        ],
        out_specs=[
            pl.BlockSpec((N_TOKENS, 1), lambda b: (0, 0)),
            pl.BlockSpec((1, 1), lambda b: (0, 0)),
        ],
        out_shape=[
            jax.ShapeDtypeStruct((N_TOKENS, 1), jnp.float32),
            jax.ShapeDtypeStruct((1, 1), jnp.float32),
        ],
        scratch_shapes=[
            pltpu.VMEM((N_TOKENS, 1), jnp.float32),
            pltpu.VMEM((N_TOKENS, 1), jnp.float32),
            pltpu.VMEM((N_TOKENS, 1), jnp.float32),
            pltpu.VMEM((N_TOKENS, 1), jnp.float32),
            pltpu.VMEM((N_TOKENS, 1), jnp.float32),
            pltpu.VMEM((N_TOKENS, 1), jnp.float32),
            pltpu.VMEM((N_TOKENS, 1), jnp.float32),
            pltpu.VMEM((N_TOKENS, 2), jnp.float32),
        ],
        compiler_params=pltpu.CompilerParams(
            dimension_semantics=("arbitrary",)),
    )(x, weight, bias_p, tgt2, tail_vectors, tb2)
    return out.reshape(N_TOKENS), loss[0, 0]


# speedup vs baseline: 2.5104x; 2.5104x over previous
import jax
import jax.numpy as jnp
from jax.experimental import pallas as pl
from jax.experimental.pallas import tpu as pltpu

IN_FEATURES = 1024
N_CLASSES = 100000
C1 = 2000
C2 = 10000
N_TOKENS = 1024
BLK = 1024
LANES = 128
NCH = BLK // LANES
NBLK = (N_CLASSES + BLK - 1) // BLK
B_S1 = C1 // BLK
B_S2 = C2 // BLK
NEG = -1e30


def _update(m_ref, s_ref, cs):
    mx = cs[0]
    for c in cs[1:]:
        mx = jnp.maximum(mx, c)
    mo = m_ref[...]
    mn = jnp.maximum(mo, mx)
    acc = s_ref[...] * jnp.exp(mo - mn)
    for c in cs:
        acc = acc + jnp.exp(c - mn)
    s_ref[...] = acc
    m_ref[...] = mn


def _flash_kernel(x_ref, w_ref, b_ref, tgt_ref, tv_ref, tb_ref,
                  out_ref, loss_ref,
                  m0, s0, m1, s1, m2, s2, pk):
    blk = pl.program_id(0)

    @pl.when(blk == 0)
    def _init():
        for r in (m0, m1, m2):
            r[...] = jnp.full((N_TOKENS, LANES), NEG, jnp.float32)
        for r in (s0, s1, s2, pk):
            r[...] = jnp.zeros((N_TOKENS, LANES), jnp.float32)

    logits = jax.lax.dot_general(
        x_ref[...].astype(jnp.bfloat16), w_ref[...].astype(jnp.bfloat16),
        (((1,), (1,)), ((), ())),
        preferred_element_type=jnp.float32)
    logits = logits + b_ref[0]

    cs = [logits[:, i * LANES:(i + 1) * LANES] for i in range(NCH)]
    cls = blk * BLK + jax.lax.broadcasted_iota(jnp.int32, (1, BLK), 1)
    clc = [cls[:, i * LANES:(i + 1) * LANES] for i in range(NCH)]

    tgt = tgt_ref[...]
    pk[...] = pk[...] + sum(
        jnp.where(c == tgt, v, 0.0) for c, v in zip(clc, cs))

    @pl.when(blk < B_S1)
    def _pure0():
        _update(m0, s0, cs)

    @pl.when(blk == B_S1)
    def _straddle01():
        _update(m0, s0, [jnp.where(c < C1, v, NEG) for c, v in zip(clc, cs)])
        _update(m1, s1, [jnp.where(c >= C1, v, NEG) for c, v in zip(clc, cs)])

    @pl.when((blk > B_S1) & (blk < B_S2))
    def _pure1():
        _update(m1, s1, cs)

    @pl.when(blk == B_S2)
    def _straddle12():
        _update(m1, s1, [jnp.where(c < C2, v, NEG) for c, v in zip(clc, cs)])
        _update(m2, s2, [jnp.where(c >= C2, v, NEG) for c, v in zip(clc, cs)])

    @pl.when((blk > B_S2) & (blk < NBLK - 1))
    def _pure2():
        _update(m2, s2, cs)

    @pl.when(blk == NBLK - 1)
    def _edge():
        _update(m2, s2,
                [jnp.where(c < N_CLASSES, v, NEG) for c, v in zip(clc, cs)])

    @pl.when(blk == NBLK - 1)
    def _fini():
        def lse_of(m_ref, s_ref):
            mp = m_ref[...]
            mt = jnp.max(mp, axis=1, keepdims=True)
            st = jnp.sum(s_ref[...] * jnp.exp(mp - mt), axis=1, keepdims=True)
            return mt, st

        mt0, st0 = lse_of(m0, s0)
        mt1, st1 = lse_of(m1, s1)
        mt2, st2 = lse_of(m2, s2)

        tlog = jax.lax.dot_general(
            x_ref[...], tv_ref[...], (((1,), (1,)), ((), ())),
            preferred_element_type=jnp.float32) + tb_ref[...]
        tmax = jnp.max(tlog, axis=1, keepdims=True)
        mh = jnp.maximum(mt0, tmax)
        sh = st0 * jnp.exp(mt0 - mh) + jnp.sum(jnp.exp(tlog - mh),
                                               axis=1, keepdims=True)
        lse0 = mh + jnp.log(sh)
        lse1 = mt1 + jnp.log(st1)
        lse2 = mt2 + jnp.log(st2)

        p = jnp.sum(pk[...], axis=1, keepdims=True)
        t = tgt_ref[...]
        is0 = t < C1
        is1 = (t >= C1) & (t < C2)
        head_pick = jnp.where(is0, p, jnp.where(is1, tlog[:, 0:1],
                                                tlog[:, 1:2]))
        tail_part = jnp.where(is0, 0.0, p - jnp.where(is1, lse1, lse2))
        out = head_pick - lse0 + tail_part
        out_ref[...] = out
        loss_ref[...] = jnp.zeros((1, 1), jnp.float32) - jnp.mean(out)


def kernel(x, target, weight, bias, tail_vectors, tail_bias):
    bias_p = jnp.pad(bias, (0, NBLK * BLK - N_CLASSES)).reshape(NBLK, 1, BLK)
    tgt2 = target.astype(jnp.int32).reshape(N_TOKENS, 1)
    tb2 = tail_bias.reshape(1, 2)
    out, loss = pl.pallas_call(
        _flash_kernel,
        grid=(NBLK,),
        in_specs=[
            pl.BlockSpec((N_TOKENS, IN_FEATURES), lambda b: (0, 0)),
            pl.BlockSpec((BLK, IN_FEATURES), lambda b: (b, 0)),
            pl.BlockSpec((1, 1, BLK), lambda b: (b, 0, 0)),
            pl.BlockSpec((N_TOKENS, 1), lambda b: (0, 0)),
            pl.BlockSpec((2, IN_FEATURES), lambda b: (0, 0)),
            pl.BlockSpec((1, 2), lambda b: (0, 0)),
        ],
        out_specs=[
            pl.BlockSpec((N_TOKENS, 1), lambda b: (0, 0)),
            pl.BlockSpec((1, 1), lambda b: (0, 0)),
        ],
        out_shape=[
            jax.ShapeDtypeStruct((N_TOKENS, 1), jnp.float32),
            jax.ShapeDtypeStruct((1, 1), jnp.float32),
        ],
        scratch_shapes=[
            pltpu.VMEM((N_TOKENS, LANES), jnp.float32),
            pltpu.VMEM((N_TOKENS, LANES), jnp.float32),
            pltpu.VMEM((N_TOKENS, LANES), jnp.float32),
            pltpu.VMEM((N_TOKENS, LANES), jnp.float32),
            pltpu.VMEM((N_TOKENS, LANES), jnp.float32),
            pltpu.VMEM((N_TOKENS, LANES), jnp.float32),
            pltpu.VMEM((N_TOKENS, LANES), jnp.float32),
        ],
        compiler_params=pltpu.CompilerParams(
            dimension_semantics=("arbitrary",)),
    )(x, weight, bias_p, tgt2, tail_vectors, tb2)
    return out.reshape(N_TOKENS), loss[0, 0]
